# baseline (speedup 1.0000x reference)
import jax
import jax.numpy as jnp
from jax.experimental import pallas as pl
from jax.experimental.pallas import tpu as pltpu


_H = 256
_B = 32
_T = 128
_CLEN = 16
_CE = 64
_HC = 128
_TC = 8
_NEG = 1e10


def _char_encode_kernel(cid_ref, tm_ref, mcomb_ref, bc_ref, ch_ref):
    cid = cid_ref[...]
    rt = cid.shape[0]

    iota_c = jax.lax.broadcasted_iota(jnp.int32, (1, 1, 128), 2)
    oh_c = (cid[:, :, None] == iota_c).astype(jnp.bfloat16)
    hall = jax.lax.dot_general(oh_c, mcomb_ref[...],
                               dimension_numbers=(((2,), (0,)), ((), ())),
                               preferred_element_type=jnp.float32)
    a = hall[:, :, :_HC]
    bmid = hall[:, :, _HC:2 * _HC]
    c = hall[:, :, 2 * _HC:]
    z = jnp.zeros((rt, 1, _HC), jnp.float32)
    conv = (bmid
            + jnp.concatenate([z, a[:, :-1, :]], axis=1)
            + jnp.concatenate([c[:, 1:, :], z], axis=1))
    h = jax.nn.relu(conv + bc_ref[...])
    cm = (cid > 0).astype(jnp.float32)[:, :, None]
    h = h - (1.0 - cm) * _NEG
    ch_ref[...] = (jnp.max(h, axis=1) * tm_ref[...]).astype(jnp.bfloat16)


def _char_encode(cids, tok_mask, mcomb, bc, rt=256):
    n = cids.shape[0]
    nt = n // rt
    return pl.pallas_call(
        _char_encode_kernel,
        out_shape=jax.ShapeDtypeStruct((n, _HC), jnp.bfloat16),
        grid_spec=pltpu.PrefetchScalarGridSpec(
            num_scalar_prefetch=0,
            grid=(2, nt // 2),
            in_specs=[
                pl.BlockSpec((rt, _CLEN), lambda c, i: (c * (nt // 2) + i, 0)),
                pl.BlockSpec((rt, 1), lambda c, i: (c * (nt // 2) + i, 0)),
                pl.BlockSpec((128, 3 * _HC), lambda c, i: (0, 0)),
                pl.BlockSpec((1, _HC), lambda c, i: (0, 0)),
            ],
            out_specs=pl.BlockSpec((rt, _HC), lambda c, i: (c * (nt // 2) + i, 0)),
        ),
        compiler_params=pltpu.CompilerParams(
            dimension_semantics=("parallel", "arbitrary")),
    )(cids, tok_mask, mcomb, bc)


def _proj0_kernel(pid_ref, wx_ref, ch_ref, ww_ref, wch_ref, mpos_ref, b_ref,
                  gx_ref):
    iota_p = jax.lax.broadcasted_iota(jnp.int32, (1, 64), 1)
    pid = jnp.swapaxes(pid_ref[0], 0, 1)
    oh_p = (pid == iota_p).astype(jnp.bfloat16)

    gx_ref[...] = (
        jnp.dot(wx_ref[...].astype(jnp.bfloat16), ww_ref[...],
                preferred_element_type=jnp.float32)
        + jnp.dot(ch_ref[...], wch_ref[...], preferred_element_type=jnp.float32)
        + jnp.dot(oh_p, mpos_ref[...], preferred_element_type=jnp.float32)
        + b_ref[...]).astype(jnp.bfloat16)


def _proj0(pids3, word_x, ch, ww, wch, mpos, b, rt=512):
    n = word_x.shape[0]
    f = ww.shape[1]
    nt = n // rt
    return pl.pallas_call(
        _proj0_kernel,
        out_shape=jax.ShapeDtypeStruct((n, f), jnp.bfloat16),
        grid_spec=pltpu.PrefetchScalarGridSpec(
            num_scalar_prefetch=0,
            grid=(2, nt // 2),
            in_specs=[
                pl.BlockSpec((1, 1, rt), lambda c, i: (c * (nt // 2) + i, 0, 0)),
                pl.BlockSpec((rt, word_x.shape[1]), lambda c, i: (c * (nt // 2) + i, 0)),
                pl.BlockSpec((rt, _HC), lambda c, i: (c * (nt // 2) + i, 0)),
                pl.BlockSpec((128, f), lambda c, i: (0, 0)),
                pl.BlockSpec((_HC, f), lambda c, i: (0, 0)),
                pl.BlockSpec((64, f), lambda c, i: (0, 0)),
                pl.BlockSpec((1, f), lambda c, i: (0, 0)),
            ],
            out_specs=pl.BlockSpec((rt, f), lambda c, i: (c * (nt // 2) + i, 0)),
        ),
        compiler_params=pltpu.CompilerParams(
            dimension_semantics=("parallel", "arbitrary")),
    )(pids3, word_x, ch, ww, wch, mpos, b)


def _linear2_kernel(xf_ref, xb_ref, wf_ref, wb_ref, b_ref, o_ref):
    o_ref[...] = (
        jnp.dot(xf_ref[...], wf_ref[...], preferred_element_type=jnp.float32)
        + jnp.dot(xb_ref[...], wb_ref[...], preferred_element_type=jnp.float32)
        + b_ref[...]).astype(jnp.bfloat16)


def _row_linear2(xf, xb, wf, wb, b, rt=512):
    n, d = xf.shape
    f = wf.shape[1]
    nt = n // rt
    return pl.pallas_call(
        _linear2_kernel,
        out_shape=jax.ShapeDtypeStruct((n, f), jnp.bfloat16),
        grid_spec=pltpu.PrefetchScalarGridSpec(
            num_scalar_prefetch=0,
            grid=(2, nt // 2),
            in_specs=[pl.BlockSpec((rt, d), lambda c, i: (c * (nt // 2) + i, 0)),
                      pl.BlockSpec((rt, d), lambda c, i: (c * (nt // 2) + i, 0)),
                      pl.BlockSpec((d, f), lambda c, i: (0, 0)),
                      pl.BlockSpec((d, f), lambda c, i: (0, 0)),
                      pl.BlockSpec((1, f), lambda c, i: (0, 0))],
            out_specs=pl.BlockSpec((rt, f), lambda c, i: (c * (nt // 2) + i, 0)),
        ),
        compiler_params=pltpu.CompilerParams(
            dimension_semantics=("parallel", "arbitrary")),
    )(xf, xb, wf, wb, b)


def _lstm_kernel(gxf_ref, gxb_ref, m_ref, mb_ref, whf_ref, whb_ref,
                 hf_ref, hb_ref, hf_scr, cf_scr, hb_scr, cb_scr):
    @pl.when(pl.program_id(0) == 0)
    def _():
        hf_scr[...] = jnp.zeros_like(hf_scr)
        cf_scr[...] = jnp.zeros_like(cf_scr)
        hb_scr[...] = jnp.zeros_like(hb_scr)
        cb_scr[...] = jnp.zeros_like(cb_scr)

    whf = whf_ref[...]
    whb = whb_ref[...]

    def step(gates, c_prev, h_prev, m_t):
        i_g = jax.nn.sigmoid(gates[:, 0 * _H:1 * _H])
        f_g = jax.nn.sigmoid(gates[:, 1 * _H:2 * _H])
        g_g = jnp.tanh(gates[:, 2 * _H:3 * _H])
        o_g = jax.nn.sigmoid(gates[:, 3 * _H:4 * _H])
        c_new = f_g * c_prev + i_g * g_g
        h_new = o_g * jnp.tanh(c_new)
        valid = m_t > 0.0
        return jnp.where(valid, c_new, c_prev), jnp.where(valid, h_new, h_prev)

    for j in range(_TC):
        jb = _TC - 1 - j
        gates_f = gxf_ref[:, j] + jnp.dot(
            hf_scr[...].astype(jnp.bfloat16), whf,
            preferred_element_type=jnp.float32)
        gates_b = gxb_ref[:, jb] + jnp.dot(
            hb_scr[...].astype(jnp.bfloat16), whb,
            preferred_element_type=jnp.float32)
        mf_t = m_ref[:, j]
        mb_t = mb_ref[:, jb]
        cf, hf = step(gates_f, cf_scr[...], hf_scr[...], mf_t)
        cb, hb = step(gates_b, cb_scr[...], hb_scr[...], mb_t)
        cf_scr[...] = cf
        hf_scr[...] = hf
        cb_scr[...] = cb
        hb_scr[...] = hb
        hf_ref[:, j] = (hf * mf_t).astype(jnp.bfloat16)
        hb_ref[:, jb] = (hb * mb_t).astype(jnp.bfloat16)


def _bilstm_layer(gx, mask_bt1, whh_f, whh_b):
    n_chunks = _T // _TC
    rev = n_chunks - 1
    out = jax.ShapeDtypeStruct((_B, _T, _H), jnp.bfloat16)
    return pl.pallas_call(
        _lstm_kernel,
        out_shape=[out, out],
        grid_spec=pltpu.PrefetchScalarGridSpec(
            num_scalar_prefetch=0,
            grid=(n_chunks,),
            in_specs=[
                pl.BlockSpec((_B, _TC, 4 * _H), lambda i: (0, i, 0)),
                pl.BlockSpec((_B, _TC, 4 * _H), lambda i: (0, rev - i, 1)),
                pl.BlockSpec((_B, _TC, 1), lambda i: (0, i, 0)),
                pl.BlockSpec((_B, _TC, 1), lambda i: (0, rev - i, 0)),
                pl.BlockSpec((_H, 4 * _H), lambda i: (0, 0)),
                pl.BlockSpec((_H, 4 * _H), lambda i: (0, 0)),
            ],
            out_specs=[pl.BlockSpec((_B, _TC, _H), lambda i: (0, i, 0)),
                       pl.BlockSpec((_B, _TC, _H), lambda i: (0, rev - i, 0))],
            scratch_shapes=[pltpu.VMEM((_B, _H), jnp.float32)] * 4,
        ),
        compiler_params=pltpu.CompilerParams(
            dimension_semantics=("arbitrary",)),
    )(gx, gx, mask_bt1, mask_bt1, whh_f, whh_b)


def _head_kernel(hf_ref, hb_ref, m_ref, hsse_ref, w1tok_ref, w1sent_ref,
                 w1sse_ref, b1_ref, w2_ref, b2_ref,
                 o0_ref, o1_ref, o2_ref, o3_ref):
    hid = jnp.concatenate([hf_ref[...], hb_ref[...]], axis=2)
    m = m_ref[...]

    sent = jnp.max(hid.astype(jnp.float32) - (1.0 - m) * _NEG,
                   axis=1).astype(jnp.bfloat16)
    bias = (jnp.dot(sent, w1sent_ref[...], preferred_element_type=jnp.float32)
            + jnp.dot(hsse_ref[...], w1sse_ref[...],
                      preferred_element_type=jnp.float32)
            + b1_ref[...])

    h1 = jax.lax.dot_general(hid, w1tok_ref[...],
                             dimension_numbers=(((2,), (0,)), ((), ())),
                             preferred_element_type=jnp.float32)
    h1 = jax.nn.relu(h1 + bias[:, None, :]).astype(jnp.bfloat16)
    out = (jax.lax.dot_general(h1, w2_ref[...],
                               dimension_numbers=(((2,), (0,)), ((), ())),
                               preferred_element_type=jnp.float32)
           + b2_ref[...])
    o0_ref[...] = out[:, :, 0]
    o1_ref[...] = out[:, :, 1]
    o2_ref[...] = out[:, :, 2]
    o3_ref[...] = out[:, :, 3]


def _fused_heads(hf, hb, mask_bt1, hsse, subj_w1, subj_b1, subj_w2,
                 subj_b2, obj_w1, obj_b1, obj_w2, obj_b2, bb=8):
    H2 = 2 * _H
    w1s_tok, w1s_sent = subj_w1[:H2], subj_w1[H2:]
    w1o_tok = obj_w1[:H2]
    w1o_sent = obj_w1[H2:2 * H2]
    w1o_sse = obj_w1[2 * H2:]

    w1_tok = jnp.concatenate([w1s_tok, w1o_tok], axis=1).astype(jnp.bfloat16)
    w1_sent = jnp.concatenate([w1s_sent, w1o_sent], axis=1).astype(jnp.bfloat16)
    w1_sse = jnp.concatenate(
        [jnp.zeros((2 * H2, H2), jnp.float32), w1o_sse],
        axis=1).astype(jnp.bfloat16)
    b1 = jnp.concatenate([subj_b1, obj_b1], axis=1)
    w2 = jnp.concatenate(
        [jnp.concatenate([subj_w2, jnp.zeros((H2, 2), jnp.float32)], axis=1),
         jnp.concatenate([jnp.zeros((H2, 2), jnp.float32), obj_w2], axis=1)],
        axis=0).astype(jnp.bfloat16)
    b2 = jnp.concatenate([subj_b2, obj_b2], axis=1)

    nb = _B // bb
    oshape = jax.ShapeDtypeStruct((_B, _T), jnp.float32)
    return pl.pallas_call(
        _head_kernel,
        out_shape=[oshape] * 4,
        grid_spec=pltpu.PrefetchScalarGridSpec(
            num_scalar_prefetch=0,
            grid=(2, nb // 2),
            in_specs=[
                pl.BlockSpec((bb, _T, _H), lambda c, i: (c * (nb // 2) + i, 0, 0)),
                pl.BlockSpec((bb, _T, _H), lambda c, i: (c * (nb // 2) + i, 0, 0)),
                pl.BlockSpec((bb, _T, 1), lambda c, i: (c * (nb // 2) + i, 0, 0)),
                pl.BlockSpec((bb, 2 * H2), lambda c, i: (c * (nb // 2) + i, 0)),
                pl.BlockSpec((H2, 2 * H2), lambda c, i: (0, 0)),
                pl.BlockSpec((H2, 2 * H2), lambda c, i: (0, 0)),
                pl.BlockSpec((2 * H2, 2 * H2), lambda c, i: (0, 0)),
                pl.BlockSpec((1, 2 * H2), lambda c, i: (0, 0)),
                pl.BlockSpec((2 * H2, 4), lambda c, i: (0, 0)),
                pl.BlockSpec((1, 4), lambda c, i: (0, 0)),
            ],
            out_specs=[pl.BlockSpec((bb, _T),
                                    lambda c, i: (c * (nb // 2) + i, 0))] * 4,
        ),
        compiler_params=pltpu.CompilerParams(
            dimension_semantics=("parallel", "arbitrary")),
    )(hf, hb, mask_bt1, hsse, w1_tok, w1_sent, w1_sse, b1, w2, b2)


def kernel(words, chars, pos_tags, subj_start_position, subj_end_position, mask,
           nearest_subj_position_for_each_token, distance_to_nearest_subj,
           distance_to_subj, nearest_obj_start_position_for_each_token,
           distance_to_nearest_obj_start,
           word_emb, char_emb, pos_emb, char_conv_w, char_conv_b,
           l0_fwd_wih, l0_fwd_whh, l0_fwd_b, l0_bwd_wih, l0_bwd_whh, l0_bwd_b,
           l1_fwd_wih, l1_fwd_whh, l1_fwd_b, l1_bwd_wih, l1_bwd_whh, l1_bwd_b,
           subj_w1, subj_b1, subj_w2, subj_b2, obj_w1, obj_b1, obj_w2, obj_b2):
    n = _T * _B

    cids = chars.reshape(n, _CLEN)
    pids3 = pos_tags.reshape(n // 512, 1, 512)
    tok_mask = mask.reshape(n, 1)
    mask_bt1 = mask.reshape(_B, _T, 1)

    word_x = jnp.take(word_emb.astype(jnp.bfloat16),
                      words.reshape(n), axis=0)

    wih0 = jnp.concatenate([l0_fwd_wih, l0_bwd_wih], axis=1)
    b0 = jnp.concatenate([l0_fwd_b, l0_bwd_b], axis=1)
    ww = wih0[:128].astype(jnp.bfloat16)
    wch = wih0[128:256].astype(jnp.bfloat16)
    mpos = jnp.zeros((64, 8 * _H), jnp.float32)
    mpos = mpos.at[:50].set(pos_emb @ wih0[256:384]).astype(jnp.bfloat16)
    ce_pad = jnp.zeros((128, _CE), jnp.float32).at[:100].set(char_emb)
    mcomb = jnp.concatenate(
        [ce_pad @ char_conv_w[0], ce_pad @ char_conv_w[1],
         ce_pad @ char_conv_w[2]], axis=1).astype(jnp.bfloat16)
    ch = _char_encode(cids, tok_mask, mcomb, char_conv_b)
    gx0 = _proj0(pids3, word_x, ch, ww, wch, mpos, b0)
    gx0 = gx0.reshape(_B, _T, 8 * _H)
    h0f, h0b = _bilstm_layer(gx0, mask_bt1,
                             l0_fwd_whh.astype(jnp.bfloat16),
                             l0_bwd_whh.astype(jnp.bfloat16))

    wih1 = jnp.concatenate([l1_fwd_wih, l1_bwd_wih],
                           axis=1).astype(jnp.bfloat16)
    b1 = jnp.concatenate([l1_fwd_b, l1_bwd_b], axis=1)
    gx1 = _row_linear2(h0f.reshape(n, _H), h0b.reshape(n, _H),
                       wih1[:_H], wih1[_H:], b1).reshape(_B, _T, 8 * _H)
    h1f, h1b = _bilstm_layer(gx1, mask_bt1,
                             l1_fwd_whh.astype(jnp.bfloat16),
                             l1_bwd_whh.astype(jnp.bfloat16))

    def at(h, idx):
        return jnp.take_along_axis(h, idx[:, None, None], axis=1)[:, 0]

    hsse = jnp.concatenate(
        [at(h1f, subj_start_position), at(h1b, subj_start_position),
         at(h1f, subj_end_position), at(h1b, subj_end_position)],
        axis=1)
    o = _fused_heads(h1f, h1b, mask_bt1, hsse, subj_w1, subj_b1, subj_w2,
                     subj_b2, obj_w1, obj_b1, obj_w2, obj_b2)
    return (o[0], o[1], o[2], o[3])
